# baseline (speedup 1.0000x reference)
import functools
import math

import jax
import jax.numpy as jnp
from jax import lax
from jax.experimental import pallas as pl
from jax.experimental.pallas import tpu as pltpu
from jax.experimental.pallas import tpu_sc as plsc

N_CLASS = 1000
BATCH = 4096
M = 0.2
S = 30.0
EPS = 0.1

_COS_M = math.cos(M)
_SIN_M = math.sin(M)
_TH = math.cos(M)
_MM = math.sin(math.pi - M) * M

BLK = 512
NBLK = BATCH // BLK

_NC = 2
_NS = 16
_NW = _NC * _NS
_RPW = BATCH // _NW
_CHUNK = 64
_W = 1024


def _aam_per_row(vals, onehot, colmask=None):
    c_lab = jnp.sum(vals * onehot, axis=1, keepdims=True)
    sine = jnp.sqrt(jnp.clip(1.0 - c_lab * c_lab, 0.0, 1.0))
    phi = c_lab * _COS_M - sine * _SIN_M
    phi = jnp.where(c_lab - _TH > 0, phi, c_lab - _MM)
    delta = S * (phi - c_lab)
    t = S * vals
    t_mod = t + onehot * delta
    if colmask is None:
        t_sum = jnp.sum(t, axis=1, keepdims=True)
    else:
        t_mod = jnp.where(colmask, t_mod, -jnp.inf)
        t_sum = jnp.sum(jnp.where(colmask, t, 0.0), axis=1, keepdims=True)
    m = jnp.max(t_mod, axis=1, keepdims=True)
    sumexp = jnp.sum(jnp.exp(t_mod - m), axis=1, keepdims=True)
    lse = m + jnp.log(sumexp)
    rmean = (t_sum + delta) / N_CLASS
    nll = lse - S * phi
    smooth = lse - rmean
    return (1.0 - EPS) * nll + EPS * smooth


def _sample_body(x_ref, lab_ref, psum_ref, cnt_ref):
    i = pl.program_id(0)
    x = x_ref[...]
    labs = lab_ref[0]
    cols = jax.lax.broadcasted_iota(jnp.int32, (BLK, N_CLASS), 1)
    onehot = (cols == labs).astype(x.dtype)
    per = _aam_per_row(x, onehot)

    @pl.when(i == 0)
    def _():
        psum_ref[...] = jnp.zeros_like(psum_ref)
        cnt_ref[...] = jnp.zeros_like(cnt_ref)

    psum_ref[...] += jnp.sum(per, axis=0, keepdims=True)
    cnt_ref[...] += jnp.sum(onehot, axis=0, keepdims=True)


def _center_body(sums_ref, cntcol_ref, csum_ref, npres_ref):
    sums = sums_ref[...]
    cnt = cntcol_ref[...]
    present = cnt > 0
    inv = jnp.where(present, 1.0 / jnp.where(present, cnt, 1.0), 0.0)
    centers = sums * inv
    rows = jax.lax.broadcasted_iota(jnp.int32, (N_CLASS, _W), 0)
    cols = jax.lax.broadcasted_iota(jnp.int32, (N_CLASS, _W), 1)
    diag = (rows == cols).astype(sums.dtype)
    per = _aam_per_row(centers, diag, colmask=cols < N_CLASS)
    per = jnp.where(present, per, 0.0)
    csum_ref[...] = jnp.sum(per, axis=0, keepdims=True)
    npres_ref[...] = jnp.sum(present.astype(jnp.float32), axis=0, keepdims=True)


_CSLC = _W // _NW
_RCHUNK = 1024
_L = 16


def _sc_scatter_body(xt_hbm, lab_hbm, out_hbm, labs_v, stage_v, acc_v):
    c = lax.axis_index("c")
    s = lax.axis_index("s")
    wid = s * _NC + c
    c0 = wid * _CSLC

    def zrow(i, carry):
        acc_v[i, pl.ds(0, _L)] = jnp.zeros((_L,), jnp.float32)
        acc_v[i, pl.ds(_L, _L)] = jnp.zeros((_L,), jnp.float32)
        return carry

    lax.fori_loop(0, N_CLASS, zrow, 0)

    pltpu.sync_copy(lab_hbm, labs_v)

    iota = lax.iota(jnp.int32, _L)
    for k in range(BATCH // _RCHUNK):
        pltpu.sync_copy(
            xt_hbm.at[wid, pl.ds(k * _RCHUNK, _RCHUNK), :], stage_v
        )

        def grp(g, carry):
            lr = g * _L
            labs16 = labs_v[pl.ds(k * _RCHUNK + lr, _L)]
            rows = lr + iota
            for col in range(_CSLC):
                cvec = jnp.full((_L,), col, jnp.int32)
                vals = plsc.load_gather(stage_v, [rows, cvec])
                plsc.addupdate_scatter(acc_v, [labs16, cvec], vals)
            return carry

        lax.fori_loop(0, _RCHUNK // _L, grp, 0)

    pltpu.sync_copy(acc_v, out_hbm.at[wid])


_sc_scatter = functools.partial(
    pl.kernel,
    mesh=plsc.VectorSubcoreMesh(core_axis_name="c", subcore_axis_name="s"),
    compiler_params=pltpu.CompilerParams(
        use_tc_tiling_on_sc=False, needs_layout_passes=False
    ),
    out_type=jax.ShapeDtypeStruct((_NW, N_CLASS, _CSLC), jnp.float32),
    scratch_types=[
        pltpu.VMEM((BATCH,), jnp.int32),
        pltpu.VMEM((_RCHUNK, _CSLC), jnp.float32),
        pltpu.VMEM((N_CLASS, _CSLC), jnp.float32),
    ],
)(_sc_scatter_body)


def kernel(x, label):
    xpad = jnp.pad(x, ((0, 0), (0, _W - N_CLASS)))
    xt = xpad.reshape(BATCH, _NW, _CSLC).transpose(1, 0, 2)
    sums_t = _sc_scatter(xt, label)
    sums = sums_t.transpose(1, 0, 2).reshape(N_CLASS, _W)

    lab3 = label.reshape(NBLK, BLK, 1)
    psum, cnt = pl.pallas_call(
        _sample_body,
        grid=(NBLK,),
        in_specs=[
            pl.BlockSpec((BLK, N_CLASS), lambda i: (i, 0)),
            pl.BlockSpec((1, BLK, 1), lambda i: (i, 0, 0)),
        ],
        out_specs=[
            pl.BlockSpec((1, 1), lambda i: (0, 0)),
            pl.BlockSpec((1, N_CLASS), lambda i: (0, 0)),
        ],
        out_shape=[
            jax.ShapeDtypeStruct((1, 1), jnp.float32),
            jax.ShapeDtypeStruct((1, N_CLASS), jnp.float32),
        ],
    )(x, lab3)

    cntcol = cnt.reshape(N_CLASS, 1)
    csum, npres = pl.pallas_call(
        _center_body,
        out_shape=[
            jax.ShapeDtypeStruct((1, 1), jnp.float32),
            jax.ShapeDtypeStruct((1, 1), jnp.float32),
        ],
    )(sums, cntcol)

    loss = 0.5 * psum[0, 0] / BATCH + 0.5 * csum[0, 0] / npres[0, 0]
    return loss


# speedup vs baseline: 1.7563x; 1.7563x over previous
import functools
import math

import jax
import jax.numpy as jnp
from jax import lax
from jax.experimental import pallas as pl
from jax.experimental.pallas import tpu as pltpu
from jax.experimental.pallas import tpu_sc as plsc

N_CLASS = 1000
BATCH = 4096
M = 0.2
S = 30.0
EPS = 0.1

_COS_M = math.cos(M)
_SIN_M = math.sin(M)
_TH = math.cos(M)
_MM = math.sin(math.pi - M) * M

BLK = 512
NBLK = BATCH // BLK

_NC = 2
_NS = 16
_NW = _NC * _NS
_RPW = BATCH // _NW
_CHUNK = 64
_W = 1024


def _aam_per_row(vals, onehot, colmask=None):
    c_lab = jnp.sum(vals * onehot, axis=1, keepdims=True)
    sine = jnp.sqrt(jnp.clip(1.0 - c_lab * c_lab, 0.0, 1.0))
    phi = c_lab * _COS_M - sine * _SIN_M
    phi = jnp.where(c_lab - _TH > 0, phi, c_lab - _MM)
    delta = S * (phi - c_lab)
    t = S * vals
    t_mod = t + onehot * delta
    if colmask is None:
        t_sum = jnp.sum(t, axis=1, keepdims=True)
    else:
        t_mod = jnp.where(colmask, t_mod, -jnp.inf)
        t_sum = jnp.sum(jnp.where(colmask, t, 0.0), axis=1, keepdims=True)
    m = jnp.max(t_mod, axis=1, keepdims=True)
    sumexp = jnp.sum(jnp.exp(t_mod - m), axis=1, keepdims=True)
    lse = m + jnp.log(sumexp)
    rmean = (t_sum + delta) / N_CLASS
    nll = lse - S * phi
    smooth = lse - rmean
    return (1.0 - EPS) * nll + EPS * smooth


def _sample_body(x_ref, lab_ref, psum_ref, cnt_ref):
    i = pl.program_id(0)
    x = x_ref[...]
    labs = lab_ref[0]
    cols = jax.lax.broadcasted_iota(jnp.int32, (BLK, N_CLASS), 1)
    onehot = (cols == labs).astype(x.dtype)
    per = _aam_per_row(x, onehot)

    @pl.when(i == 0)
    def _():
        psum_ref[...] = jnp.zeros_like(psum_ref)
        cnt_ref[...] = jnp.zeros_like(cnt_ref)

    psum_ref[...] += jnp.sum(per, axis=0, keepdims=True)
    cnt_ref[...] += jnp.sum(onehot, axis=0, keepdims=True)


def _center_body(sums_ref, cntcol_ref, csum_ref, npres_ref):
    sums = sums_ref[...]
    cnt = cntcol_ref[...]
    present = cnt > 0
    inv = jnp.where(present, 1.0 / jnp.where(present, cnt, 1.0), 0.0)
    centers = sums * inv
    rows = jax.lax.broadcasted_iota(jnp.int32, (N_CLASS, _W), 0)
    cols = jax.lax.broadcasted_iota(jnp.int32, (N_CLASS, _W), 1)
    diag = (rows == cols).astype(sums.dtype)
    per = _aam_per_row(centers, diag, colmask=cols < N_CLASS)
    per = jnp.where(present, per, 0.0)
    csum_ref[...] = jnp.sum(per, axis=0, keepdims=True)
    npres_ref[...] = jnp.sum(present.astype(jnp.float32), axis=0, keepdims=True)


_CSLC = _W // _NW
_RCHUNK = 1024
_L = 16


def _sc_scatter_body(xt_hbm, lab_hbm, out_hbm, labs_v, stage_v, acc_v):
    c = lax.axis_index("c")
    s = lax.axis_index("s")
    wid = s * _NC + c
    c0 = wid * _CSLC

    def zrow(i, carry):
        acc_v[i, pl.ds(0, _L)] = jnp.zeros((_L,), jnp.float32)
        acc_v[i, pl.ds(_L, _L)] = jnp.zeros((_L,), jnp.float32)
        return carry

    lax.fori_loop(0, N_CLASS, zrow, 0)

    pltpu.sync_copy(lab_hbm, labs_v)

    iota = lax.iota(jnp.int32, _L)
    for k in range(BATCH // _RCHUNK):
        pltpu.sync_copy(
            xt_hbm.at[wid, pl.ds(k * _RCHUNK, _RCHUNK), :], stage_v
        )

        def grp(g, carry):
            lr = g * _L
            labs16 = labs_v[pl.ds(k * _RCHUNK + lr, _L)]
            rows = lr + iota
            for col in range(_CSLC):
                cvec = (iota + col) & (_CSLC - 1)
                vals = plsc.load_gather(stage_v, [rows, cvec])
                plsc.addupdate_scatter(acc_v, [labs16, cvec], vals)
            return carry

        lax.fori_loop(0, _RCHUNK // _L, grp, 0)

    pltpu.sync_copy(acc_v, out_hbm.at[wid])


_sc_scatter = functools.partial(
    pl.kernel,
    mesh=plsc.VectorSubcoreMesh(core_axis_name="c", subcore_axis_name="s"),
    compiler_params=pltpu.CompilerParams(
        use_tc_tiling_on_sc=False, needs_layout_passes=False
    ),
    out_type=jax.ShapeDtypeStruct((_NW, N_CLASS, _CSLC), jnp.float32),
    scratch_types=[
        pltpu.VMEM((BATCH,), jnp.int32),
        pltpu.VMEM((_RCHUNK, _CSLC), jnp.float32),
        pltpu.VMEM((N_CLASS, _CSLC), jnp.float32),
    ],
)(_sc_scatter_body)


def kernel(x, label):
    xpad = jnp.pad(x, ((0, 0), (0, _W - N_CLASS)))
    xt = xpad.reshape(BATCH, _NW, _CSLC).transpose(1, 0, 2)
    sums_t = _sc_scatter(xt, label)
    sums = sums_t.transpose(1, 0, 2).reshape(N_CLASS, _W)

    lab3 = label.reshape(NBLK, BLK, 1)
    psum, cnt = pl.pallas_call(
        _sample_body,
        grid=(NBLK,),
        in_specs=[
            pl.BlockSpec((BLK, N_CLASS), lambda i: (i, 0)),
            pl.BlockSpec((1, BLK, 1), lambda i: (i, 0, 0)),
        ],
        out_specs=[
            pl.BlockSpec((1, 1), lambda i: (0, 0)),
            pl.BlockSpec((1, N_CLASS), lambda i: (0, 0)),
        ],
        out_shape=[
            jax.ShapeDtypeStruct((1, 1), jnp.float32),
            jax.ShapeDtypeStruct((1, N_CLASS), jnp.float32),
        ],
    )(x, lab3)

    cntcol = cnt.reshape(N_CLASS, 1)
    csum, npres = pl.pallas_call(
        _center_body,
        out_shape=[
            jax.ShapeDtypeStruct((1, 1), jnp.float32),
            jax.ShapeDtypeStruct((1, 1), jnp.float32),
        ],
    )(sums, cntcol)

    loss = 0.5 * psum[0, 0] / BATCH + 0.5 * csum[0, 0] / npres[0, 0]
    return loss


# speedup vs baseline: 1.9419x; 1.1057x over previous
import functools
import math

import jax
import jax.numpy as jnp
from jax import lax
from jax.experimental import pallas as pl
from jax.experimental.pallas import tpu as pltpu
from jax.experimental.pallas import tpu_sc as plsc

N_CLASS = 1000
BATCH = 4096
M = 0.2
S = 30.0
EPS = 0.1

_COS_M = math.cos(M)
_SIN_M = math.sin(M)
_TH = math.cos(M)
_MM = math.sin(math.pi - M) * M

BLK = 512
NBLK = BATCH // BLK

_NC = 2
_NS = 16
_NW = _NC * _NS
_RPW = BATCH // _NW
_CHUNK = 64
_W = 1024


def _aam_per_row(vals, onehot, colmask=None):
    c_lab = jnp.sum(vals * onehot, axis=1, keepdims=True)
    sine = jnp.sqrt(jnp.clip(1.0 - c_lab * c_lab, 0.0, 1.0))
    phi = c_lab * _COS_M - sine * _SIN_M
    phi = jnp.where(c_lab - _TH > 0, phi, c_lab - _MM)
    delta = S * (phi - c_lab)
    t = S * vals
    t_mod = t + onehot * delta
    if colmask is None:
        t_sum = jnp.sum(t, axis=1, keepdims=True)
    else:
        t_mod = jnp.where(colmask, t_mod, -jnp.inf)
        t_sum = jnp.sum(jnp.where(colmask, t, 0.0), axis=1, keepdims=True)
    m = jnp.max(t_mod, axis=1, keepdims=True)
    sumexp = jnp.sum(jnp.exp(t_mod - m), axis=1, keepdims=True)
    lse = m + jnp.log(sumexp)
    rmean = (t_sum + delta) / N_CLASS
    nll = lse - S * phi
    smooth = lse - rmean
    return (1.0 - EPS) * nll + EPS * smooth


def _sample_body(x_ref, lab_ref, psum_ref, cnt_ref):
    i = pl.program_id(0)
    x = x_ref[...]
    labs = lab_ref[0]
    cols = jax.lax.broadcasted_iota(jnp.int32, (BLK, N_CLASS), 1)
    onehot = (cols == labs).astype(x.dtype)
    per = _aam_per_row(x, onehot)

    @pl.when(i == 0)
    def _():
        psum_ref[...] = jnp.zeros_like(psum_ref)
        cnt_ref[...] = jnp.zeros_like(cnt_ref)

    psum_ref[...] += jnp.sum(per, axis=0, keepdims=True)
    cnt_ref[...] += jnp.sum(onehot, axis=0, keepdims=True)


def _center_body(sums_ref, cntcol_ref, csum_ref, npres_ref):
    sums = sums_ref[...]
    cnt = cntcol_ref[...]
    present = cnt > 0
    inv = jnp.where(present, 1.0 / jnp.where(present, cnt, 1.0), 0.0)
    centers = sums * inv
    rows = jax.lax.broadcasted_iota(jnp.int32, (N_CLASS, _W), 0)
    cols = jax.lax.broadcasted_iota(jnp.int32, (N_CLASS, _W), 1)
    diag = (rows == cols).astype(sums.dtype)
    per = _aam_per_row(centers, diag, colmask=cols < N_CLASS)
    per = jnp.where(present, per, 0.0)
    csum_ref[...] = jnp.sum(per, axis=0, keepdims=True)
    npres_ref[...] = jnp.sum(present.astype(jnp.float32), axis=0, keepdims=True)


_CSLC = _W // _NW
_RCHUNK = 1024
_L = 16


def _sc_scatter_body(xt_hbm, lab_hbm, out_hbm, labs_v, stage_v, acc_v):
    c = lax.axis_index("c")
    s = lax.axis_index("s")
    wid = s * _NC + c
    c0 = wid * _CSLC

    def zrow(i, carry):
        acc_v[i, pl.ds(0, _L)] = jnp.zeros((_L,), jnp.float32)
        acc_v[i, pl.ds(_L, _L)] = jnp.zeros((_L,), jnp.float32)
        return carry

    lax.fori_loop(0, N_CLASS, zrow, 0)

    pltpu.sync_copy(lab_hbm, labs_v)

    iota = lax.iota(jnp.int32, _L)
    for k in range(BATCH // _RCHUNK):
        pltpu.sync_copy(
            xt_hbm.at[wid, pl.ds(k * _RCHUNK, _RCHUNK), :], stage_v
        )

        def grp(g, carry):
            lr = g * _L
            labs16 = labs_v[pl.ds(k * _RCHUNK + lr, _L)]
            rows = lr + iota
            for c0 in range(0, _CSLC, _L):
                vals = []
                for col in range(c0, c0 + _L):
                    cvec = (iota + col) & (_CSLC - 1)
                    vals.append(plsc.load_gather(stage_v, [rows, cvec]))
                for j, col in enumerate(range(c0, c0 + _L)):
                    cvec = (iota + col) & (_CSLC - 1)
                    plsc.addupdate_scatter(acc_v, [labs16, cvec], vals[j])
            return carry

        lax.fori_loop(0, _RCHUNK // _L, grp, 0, unroll=2)

    pltpu.sync_copy(acc_v, out_hbm.at[wid])


_sc_scatter = functools.partial(
    pl.kernel,
    mesh=plsc.VectorSubcoreMesh(core_axis_name="c", subcore_axis_name="s"),
    compiler_params=pltpu.CompilerParams(
        use_tc_tiling_on_sc=False, needs_layout_passes=False
    ),
    out_type=jax.ShapeDtypeStruct((_NW, N_CLASS, _CSLC), jnp.float32),
    scratch_types=[
        pltpu.VMEM((BATCH,), jnp.int32),
        pltpu.VMEM((_RCHUNK, _CSLC), jnp.float32),
        pltpu.VMEM((N_CLASS, _CSLC), jnp.float32),
    ],
)(_sc_scatter_body)


def kernel(x, label):
    xpad = jnp.pad(x, ((0, 0), (0, _W - N_CLASS)))
    xt = xpad.reshape(BATCH, _NW, _CSLC).transpose(1, 0, 2)
    sums_t = _sc_scatter(xt, label)
    sums = sums_t.transpose(1, 0, 2).reshape(N_CLASS, _W)

    lab3 = label.reshape(NBLK, BLK, 1)
    psum, cnt = pl.pallas_call(
        _sample_body,
        grid=(NBLK,),
        in_specs=[
            pl.BlockSpec((BLK, N_CLASS), lambda i: (i, 0)),
            pl.BlockSpec((1, BLK, 1), lambda i: (i, 0, 0)),
        ],
        out_specs=[
            pl.BlockSpec((1, 1), lambda i: (0, 0)),
            pl.BlockSpec((1, N_CLASS), lambda i: (0, 0)),
        ],
        out_shape=[
            jax.ShapeDtypeStruct((1, 1), jnp.float32),
            jax.ShapeDtypeStruct((1, N_CLASS), jnp.float32),
        ],
    )(x, lab3)

    cntcol = cnt.reshape(N_CLASS, 1)
    csum, npres = pl.pallas_call(
        _center_body,
        out_shape=[
            jax.ShapeDtypeStruct((1, 1), jnp.float32),
            jax.ShapeDtypeStruct((1, 1), jnp.float32),
        ],
    )(sums, cntcol)

    loss = 0.5 * psum[0, 0] / BATCH + 0.5 * csum[0, 0] / npres[0, 0]
    return loss


# speedup vs baseline: 2.9301x; 1.5089x over previous
import functools
import math

import jax
import jax.numpy as jnp
from jax import lax
from jax.experimental import pallas as pl
from jax.experimental.pallas import tpu as pltpu
from jax.experimental.pallas import tpu_sc as plsc

N_CLASS = 1000
BATCH = 4096
M = 0.2
S = 30.0
EPS = 0.1

_COS_M = math.cos(M)
_SIN_M = math.sin(M)
_TH = math.cos(M)
_MM = math.sin(math.pi - M) * M

BLK = 512
NBLK = BATCH // BLK

_NC = 2
_NS = 16
_NW = _NC * _NS
_RPW = BATCH // _NW
_CHUNK = 64
_W = 1024


def _aam_per_row(vals, onehot, colmask=None):
    c_lab = jnp.sum(vals * onehot, axis=1, keepdims=True)
    sine = jnp.sqrt(jnp.clip(1.0 - c_lab * c_lab, 0.0, 1.0))
    phi = c_lab * _COS_M - sine * _SIN_M
    phi = jnp.where(c_lab - _TH > 0, phi, c_lab - _MM)
    delta = S * (phi - c_lab)
    t = S * vals
    t_mod = t + onehot * delta
    if colmask is None:
        t_sum = jnp.sum(t, axis=1, keepdims=True)
    else:
        t_mod = jnp.where(colmask, t_mod, -jnp.inf)
        t_sum = jnp.sum(jnp.where(colmask, t, 0.0), axis=1, keepdims=True)
    m = jnp.max(t_mod, axis=1, keepdims=True)
    sumexp = jnp.sum(jnp.exp(t_mod - m), axis=1, keepdims=True)
    lse = m + jnp.log(sumexp)
    rmean = (t_sum + delta) / N_CLASS
    nll = lse - S * phi
    smooth = lse - rmean
    return (1.0 - EPS) * nll + EPS * smooth


def _sample_body(x_ref, lab_ref, psum_ref, cnt_ref):
    i = pl.program_id(0)
    x = x_ref[...]
    labs = lab_ref[0]
    cols = jax.lax.broadcasted_iota(jnp.int32, (BLK, N_CLASS), 1)
    onehot = (cols == labs).astype(x.dtype)
    per = _aam_per_row(x, onehot)

    @pl.when(i == 0)
    def _():
        psum_ref[...] = jnp.zeros_like(psum_ref)
        cnt_ref[...] = jnp.zeros_like(cnt_ref)

    psum_ref[...] += jnp.sum(per, axis=0, keepdims=True)
    cnt_ref[...] += jnp.sum(onehot, axis=0, keepdims=True)


def _center_body(sums_ref, cntcol_ref, csum_ref, npres_ref):
    sums = sums_ref[...]
    cnt = cntcol_ref[...]
    present = cnt > 0
    inv = jnp.where(present, 1.0 / jnp.where(present, cnt, 1.0), 0.0)
    centers = sums * inv
    rows = jax.lax.broadcasted_iota(jnp.int32, (N_CLASS, _W), 0)
    cols = jax.lax.broadcasted_iota(jnp.int32, (N_CLASS, _W), 1)
    diag = (rows == cols).astype(sums.dtype)
    per = _aam_per_row(centers, diag, colmask=cols < N_CLASS)
    per = jnp.where(present, per, 0.0)
    csum_ref[...] = jnp.sum(per, axis=0, keepdims=True)
    npres_ref[...] = jnp.sum(present.astype(jnp.float32), axis=0, keepdims=True)


_CSLC = _W // _NW
_RCHUNK = 1024
_L = 16


def _sc_scatter_body(x_hbm, lab_hbm, out_hbm, labs_v, stage_v, acc_v):
    c = lax.axis_index("c")
    s = lax.axis_index("s")
    wid = s * _NC + c
    c0 = wid * _CSLC

    def zrow(i, carry):
        acc_v[i, pl.ds(0, _L)] = jnp.zeros((_L,), jnp.float32)
        acc_v[i, pl.ds(_L, _L)] = jnp.zeros((_L,), jnp.float32)
        return carry

    lax.fori_loop(0, N_CLASS, zrow, 0)

    pltpu.sync_copy(lab_hbm, labs_v)

    iota = lax.iota(jnp.int32, _L)
    for k in range(BATCH // _RCHUNK):
        pltpu.sync_copy(
            x_hbm.at[pl.ds(k * _RCHUNK, _RCHUNK), pl.ds(c0, _CSLC)], stage_v
        )

        def grp(g, carry):
            lr = g * _L
            labs16 = labs_v[pl.ds(k * _RCHUNK + lr, _L)]
            rows = lr + iota
            for c0 in range(0, _CSLC, _L):
                vals = []
                for col in range(c0, c0 + _L):
                    cvec = (iota + col) & (_CSLC - 1)
                    vals.append(plsc.load_gather(stage_v, [rows, cvec]))
                for j, col in enumerate(range(c0, c0 + _L)):
                    cvec = (iota + col) & (_CSLC - 1)
                    plsc.addupdate_scatter(acc_v, [labs16, cvec], vals[j])
            return carry

        lax.fori_loop(0, _RCHUNK // _L, grp, 0, unroll=2)

    pltpu.sync_copy(acc_v, out_hbm.at[:, pl.ds(c0, _CSLC)])


_sc_scatter = functools.partial(
    pl.kernel,
    mesh=plsc.VectorSubcoreMesh(core_axis_name="c", subcore_axis_name="s"),
    compiler_params=pltpu.CompilerParams(
        use_tc_tiling_on_sc=False, needs_layout_passes=False
    ),
    out_type=jax.ShapeDtypeStruct((N_CLASS, _W), jnp.float32),
    scratch_types=[
        pltpu.VMEM((BATCH,), jnp.int32),
        pltpu.VMEM((_RCHUNK, _CSLC), jnp.float32),
        pltpu.VMEM((N_CLASS, _CSLC), jnp.float32),
    ],
)(_sc_scatter_body)


def kernel(x, label):
    xpad = jnp.pad(x, ((0, 0), (0, _W - N_CLASS)))
    sums = _sc_scatter(xpad, label)

    lab3 = label.reshape(NBLK, BLK, 1)
    psum, cnt = pl.pallas_call(
        _sample_body,
        grid=(NBLK,),
        in_specs=[
            pl.BlockSpec((BLK, N_CLASS), lambda i: (i, 0)),
            pl.BlockSpec((1, BLK, 1), lambda i: (i, 0, 0)),
        ],
        out_specs=[
            pl.BlockSpec((1, 1), lambda i: (0, 0)),
            pl.BlockSpec((1, N_CLASS), lambda i: (0, 0)),
        ],
        out_shape=[
            jax.ShapeDtypeStruct((1, 1), jnp.float32),
            jax.ShapeDtypeStruct((1, N_CLASS), jnp.float32),
        ],
    )(x, lab3)

    cntcol = cnt.reshape(N_CLASS, 1)
    csum, npres = pl.pallas_call(
        _center_body,
        out_shape=[
            jax.ShapeDtypeStruct((1, 1), jnp.float32),
            jax.ShapeDtypeStruct((1, 1), jnp.float32),
        ],
    )(sums, cntcol)

    loss = 0.5 * psum[0, 0] / BATCH + 0.5 * csum[0, 0] / npres[0, 0]
    return loss
